# baseline (speedup 1.0000x reference)
import jax
import jax.numpy as jnp
from jax import lax
from jax.experimental import pallas as pl
from jax.experimental.pallas import tpu as pltpu
from jax.experimental.pallas import tpu_sc as plsc

L = 16
NCH = 96
NPIX = 512 * 512
CHUNK = 16384
NCHUNK = NPIX // CHUNK
ROWS = CHUNK // 512
GPR = 512 // L
NBUF = 4
NG = NCHUNK // NBUF
PRE = 2
NW = 32
CPW = NCH // NW
NBIN = 256


def _body(img_hbm4, out_hbm4, b0, b1, b2, b3, hist, lut,
          si0, si1, si2, si3, so0, so1, so2, so3):
    img_hbm = img_hbm4.reshape(NCH * 512, 512)
    out_hbm = out_hbm4.reshape(NCH * 512, 512)
    bufs = (b0, b1, b2, b3)
    isems = (si0, si1, si2, si3)
    osems = (so0, so1, so2, so3)
    cid = lax.axis_index("c")
    sid = lax.axis_index("s")
    wid = sid * 2 + cid

    ones = jnp.full((L,), 1, jnp.int32)
    zeros = jnp.zeros((L,), jnp.int32)
    iota = lax.iota(jnp.int32, L)
    total = jnp.int32(NPIX)

    def islab(ch, c):
        return img_hbm.at[pl.ds(ch * 512 + c * ROWS, ROWS), :]

    def oslab(ch, c):
        return out_hbm.at[pl.ds(ch * 512 + c * ROWS, ROWS), :]

    def hist_chunk(buf):
        @plsc.parallel_loop(0, GPR, 1)
        def _(i):
            col = lax.shift_left(i, 4)
            for r in range(ROWS):
                v = jnp.bitwise_and(buf[r, pl.ds(col, L)], 255)
                plsc.addupdate_scatter(hist, [v], ones)

    def apply_chunk(buf):
        @plsc.parallel_loop(0, GPR, 1)
        def _(i):
            col = lax.shift_left(i, 4)
            for r in range(ROWS):
                v = jnp.bitwise_and(buf[r, pl.ds(col, L)], 255)
                buf[r, pl.ds(col, L)] = plsc.load_gather(lut, [v])

    def channel_body(j, _):
        ch = wid + NW * j

        for k in range(NBIN // L):
            hist[pl.ds(k * L, L)] = zeros
        for b in range(PRE):
            pltpu.async_copy(islab(ch, b), bufs[b], isems[b])

        def p1_group(g, _):
            for b in range(NBUF):
                c = g * NBUF + b
                nb = (b + PRE) % NBUF
                if b < NBUF - PRE:
                    pltpu.async_copy(islab(ch, c + PRE), bufs[nb], isems[nb])
                else:
                    @pl.when(g < NG - 1)
                    def _():
                        pltpu.async_copy(islab(ch, c + PRE), bufs[nb], isems[nb])
                pltpu.make_async_copy(islab(ch, c), bufs[b], isems[b]).wait()
                hist_chunk(bufs[b])
            return 0

        lax.fori_loop(0, NG, p1_group, 0)

        for b in range(PRE):
            pltpu.async_copy(islab(ch, b), bufs[b], isems[b])

        carry = jnp.int32(0)
        m = jnp.int32(0)
        for k in range(NBIN // L):
            h = hist[pl.ds(k * L, L)]
            csum = plsc.cumsum(h) + carry
            carry = jnp.max(csum)
            m = jnp.maximum(m, jnp.max(jnp.where(csum < total, csum, 0)))
            hist[pl.ds(k * L, L)] = csum

        step = lax.div(m, jnp.int32(255))
        half = lax.div(step, jnp.int32(2))
        sstep = jnp.maximum(step, jnp.int32(1))
        is_id = step == 0

        lut[pl.ds(0, L)] = zeros
        for k in range(NBIN // L):
            csum = hist[pl.ds(k * L, L)]
            lv = lax.div(csum + half, sstep)
            lv = jnp.clip(lv, 0, 255)
            idv = iota + (k * L + 1)
            lv = jnp.where(is_id, idv, lv)
            lut[pl.ds(k * L + 1, L)] = lv

        def p2_group(g, _):
            for b in range(NBUF):
                c = g * NBUF + b
                nb = (b + PRE) % NBUF
                if b < NBUF - PRE:
                    @pl.when(g >= 1)
                    def _():
                        pltpu.make_async_copy(
                            bufs[nb], oslab(ch, c - PRE), osems[nb]
                        ).wait()
                    pltpu.async_copy(islab(ch, c + PRE), bufs[nb], isems[nb])
                else:
                    @pl.when(g < NG - 1)
                    def _():
                        pltpu.make_async_copy(
                            bufs[nb], oslab(ch, c - PRE), osems[nb]
                        ).wait()
                        pltpu.async_copy(islab(ch, c + PRE), bufs[nb], isems[nb])
                pltpu.make_async_copy(islab(ch, c), bufs[b], isems[b]).wait()
                apply_chunk(bufs[b])
                pltpu.async_copy(bufs[b], oslab(ch, c), osems[b])
            return 0

        lax.fori_loop(0, NG, p2_group, 0)
        for b in range(NBUF):
            pltpu.make_async_copy(
                bufs[b], oslab(ch, (NG - 1) * NBUF + b), osems[b]
            ).wait()
        return 0

    lax.fori_loop(0, CPW, channel_body, 0)


def kernel(img):
    B, C, H, W = img.shape
    mesh = plsc.VectorSubcoreMesh(
        core_axis_name="c", subcore_axis_name="s", num_cores=2, num_subcores=16
    )
    out = pl.kernel(
        _body,
        out_type=jax.ShapeDtypeStruct((B, C, H, W), jnp.int32),
        mesh=mesh,
        scratch_types=[
            pltpu.VMEM((ROWS, 512), jnp.int32),
            pltpu.VMEM((ROWS, 512), jnp.int32),
            pltpu.VMEM((ROWS, 512), jnp.int32),
            pltpu.VMEM((ROWS, 512), jnp.int32),
            pltpu.VMEM((NBIN,), jnp.int32),
            pltpu.VMEM((NBIN + L,), jnp.int32),
            pltpu.SemaphoreType.DMA,
            pltpu.SemaphoreType.DMA,
            pltpu.SemaphoreType.DMA,
            pltpu.SemaphoreType.DMA,
            pltpu.SemaphoreType.DMA,
            pltpu.SemaphoreType.DMA,
            pltpu.SemaphoreType.DMA,
            pltpu.SemaphoreType.DMA,
        ],
        compiler_params=pltpu.CompilerParams(needs_layout_passes=False),
    )(img)
    return out


# speedup vs baseline: 1.1776x; 1.1776x over previous
import jax
import jax.numpy as jnp
from jax import lax
from jax.experimental import pallas as pl
from jax.experimental.pallas import tpu as pltpu
from jax.experimental.pallas import tpu_sc as plsc

L = 16
NCH = 96
NPIX = 512 * 512
CHUNK = 16384
NCHUNK = NPIX // CHUNK
ROWS = CHUNK // 512
NBUF = 4
PRE = 2
NW = 32
CPW = NCH // NW
NBIN = 256
UNROLL = 16


def _body(img_hbm4, out_hbm4, b0, b1, b2, b3, hist, lut,
          si0, si1, si2, si3, so0, so1, so2, so3):
    img_hbm = img_hbm4.reshape(NCH * 512, 512)
    out_hbm = out_hbm4.reshape(NCH * 512, 512)
    bufs = (b0, b1, b2, b3)
    isems = (si0, si1, si2, si3)
    osems = (so0, so1, so2, so3)
    cid = lax.axis_index("c")
    sid = lax.axis_index("s")
    wid = sid * 2 + cid

    ones = jnp.full((L,), 1, jnp.int32)
    zeros = jnp.zeros((L,), jnp.int32)
    iota = lax.iota(jnp.int32, L)
    total = jnp.int32(NPIX)

    def in_dma(ch, c):
        return pltpu.async_copy(
            img_hbm.at[pl.ds(ch * 512 + c * ROWS, ROWS), :],
            bufs[c % NBUF],
            isems[c % NBUF],
        )

    def out_dma(ch, c):
        return pltpu.async_copy(
            bufs[c % NBUF],
            out_hbm.at[pl.ds(ch * 512 + c * ROWS, ROWS), :],
            osems[c % NBUF],
        )

    def channel_body(j, _):
        ch = wid + NW * j

        for k in range(NBIN // L):
            hist[pl.ds(k * L, L)] = zeros
        pend = {c: in_dma(ch, c) for c in range(PRE)}
        for c in range(NCHUNK):
            n = c + PRE
            if n < NCHUNK:
                pend[n] = in_dma(ch, n)
            pend.pop(c).wait()
            buf = bufs[c % NBUF]

            @plsc.parallel_loop(0, CHUNK // L, 1, unroll=UNROLL)
            def _(i):
                r = lax.shift_right_logical(i, 5)
                col = lax.shift_left(jnp.bitwise_and(i, 31), 4)
                v = jnp.bitwise_and(buf[r, pl.ds(col, L)], 255)
                plsc.addupdate_scatter(hist, [v], ones)

        pend = {c: in_dma(ch, c) for c in range(PRE)}

        carry = jnp.int32(0)
        m = jnp.int32(0)
        for k in range(NBIN // L):
            h = hist[pl.ds(k * L, L)]
            csum = plsc.cumsum(h) + carry
            carry = jnp.max(csum)
            m = jnp.maximum(m, jnp.max(jnp.where(csum < total, csum, 0)))
            hist[pl.ds(k * L, L)] = csum

        step = lax.div(m, jnp.int32(255))
        half = lax.div(step, jnp.int32(2))
        sstep = jnp.maximum(step, jnp.int32(1))
        is_id = step == 0

        lut[pl.ds(0, L)] = zeros
        for k in range(NBIN // L):
            csum = hist[pl.ds(k * L, L)]
            lv = lax.div(csum + half, sstep)
            lv = jnp.clip(lv, 0, 255)
            idv = iota + (k * L + 1)
            lv = jnp.where(is_id, idv, lv)
            lut[pl.ds(k * L + 1, L)] = lv

        outs = {}
        for c in range(NCHUNK):
            n = c + PRE
            if n < NCHUNK:
                if n >= NBUF:
                    outs.pop(n - NBUF).wait()
                pend[n] = in_dma(ch, n)
            pend.pop(c).wait()
            buf = bufs[c % NBUF]

            @plsc.parallel_loop(0, CHUNK // L, 1, unroll=UNROLL)
            def _(i):
                r = lax.shift_right_logical(i, 5)
                col = lax.shift_left(jnp.bitwise_and(i, 31), 4)
                v = jnp.bitwise_and(buf[r, pl.ds(col, L)], 255)
                buf[r, pl.ds(col, L)] = plsc.load_gather(lut, [v])

            outs[c] = out_dma(ch, c)
        for c in sorted(outs):
            outs.pop(c).wait()
        return 0

    lax.fori_loop(0, CPW, channel_body, 0)


def kernel(img):
    B, C, H, W = img.shape
    mesh = plsc.VectorSubcoreMesh(
        core_axis_name="c", subcore_axis_name="s", num_cores=2, num_subcores=16
    )
    out = pl.kernel(
        _body,
        out_type=jax.ShapeDtypeStruct((B, C, H, W), jnp.int32),
        mesh=mesh,
        scratch_types=[
            pltpu.VMEM((ROWS, 512), jnp.int32),
            pltpu.VMEM((ROWS, 512), jnp.int32),
            pltpu.VMEM((ROWS, 512), jnp.int32),
            pltpu.VMEM((ROWS, 512), jnp.int32),
            pltpu.VMEM((NBIN,), jnp.int32),
            pltpu.VMEM((NBIN + L,), jnp.int32),
            pltpu.SemaphoreType.DMA,
            pltpu.SemaphoreType.DMA,
            pltpu.SemaphoreType.DMA,
            pltpu.SemaphoreType.DMA,
            pltpu.SemaphoreType.DMA,
            pltpu.SemaphoreType.DMA,
            pltpu.SemaphoreType.DMA,
            pltpu.SemaphoreType.DMA,
        ],
        compiler_params=pltpu.CompilerParams(needs_layout_passes=False),
    )(img)
    return out


# speedup vs baseline: 1.1849x; 1.0062x over previous
import jax
import jax.numpy as jnp
from jax import lax
from jax.experimental import pallas as pl
from jax.experimental.pallas import tpu as pltpu
from jax.experimental.pallas import tpu_sc as plsc

L = 16
NCH = 96
NPIX = 512 * 512
CHUNK = 16384
NCHUNK = NPIX // CHUNK
ROWS = CHUNK // 512
NBUF = 4
PRE = 2
NW = 32
CPW = NCH // NW
NBIN = 256
UNROLL = 8


def _body(img_hbm4, out_hbm4, b0, b1, b2, b3, hist, lut,
          si0, si1, si2, si3, so0, so1, so2, so3):
    img_hbm = img_hbm4.reshape(NCH * 512, 512)
    out_hbm = out_hbm4.reshape(NCH * 512, 512)
    bufs = (b0, b1, b2, b3)
    isems = (si0, si1, si2, si3)
    osems = (so0, so1, so2, so3)
    cid = lax.axis_index("c")
    sid = lax.axis_index("s")
    wid = sid * 2 + cid

    ones = jnp.full((L,), 1, jnp.int32)
    zeros = jnp.zeros((L,), jnp.int32)
    iota = lax.iota(jnp.int32, L)
    total = jnp.int32(NPIX)

    def in_dma(ch, c):
        return pltpu.async_copy(
            img_hbm.at[pl.ds(ch * 512 + c * ROWS, ROWS), :],
            bufs[c % NBUF],
            isems[c % NBUF],
        )

    def out_dma(ch, c):
        return pltpu.async_copy(
            bufs[c % NBUF],
            out_hbm.at[pl.ds(ch * 512 + c * ROWS, ROWS), :],
            osems[c % NBUF],
        )

    def channel_body(j, _):
        ch = wid + NW * j

        for k in range(NBIN // L):
            hist[pl.ds(k * L, L)] = zeros
        pend = {c: in_dma(ch, c) for c in range(PRE)}
        for c in range(NCHUNK):
            n = c + PRE
            if n < NCHUNK:
                pend[n] = in_dma(ch, n)
            pend.pop(c).wait()
            buf = bufs[c % NBUF]

            @plsc.parallel_loop(0, CHUNK // L, 1, unroll=UNROLL)
            def _(i):
                r = lax.shift_right_logical(i, 5)
                col = lax.shift_left(jnp.bitwise_and(i, 31), 4)
                v = jnp.bitwise_and(buf[r, pl.ds(col, L)], 255)
                plsc.addupdate_scatter(hist, [v], ones)

        pend = {c: in_dma(ch, c) for c in range(PRE)}

        carry = jnp.int32(0)
        m = jnp.int32(0)
        for k in range(NBIN // L):
            h = hist[pl.ds(k * L, L)]
            csum = plsc.cumsum(h) + carry
            carry = jnp.max(csum)
            m = jnp.maximum(m, jnp.max(jnp.where(csum < total, csum, 0)))
            hist[pl.ds(k * L, L)] = csum

        step = lax.div(m, jnp.int32(255))
        half = lax.div(step, jnp.int32(2))
        sstep = jnp.maximum(step, jnp.int32(1))
        is_id = step == 0

        lut[pl.ds(0, L)] = zeros
        for k in range(NBIN // L):
            csum = hist[pl.ds(k * L, L)]
            lv = lax.div(csum + half, sstep)
            lv = jnp.clip(lv, 0, 255)
            idv = iota + (k * L + 1)
            lv = jnp.where(is_id, idv, lv)
            lut[pl.ds(k * L + 1, L)] = lv

        outs = {}
        for c in range(NCHUNK):
            n = c + PRE
            if n < NCHUNK:
                if n >= NBUF:
                    outs.pop(n - NBUF).wait()
                pend[n] = in_dma(ch, n)
            pend.pop(c).wait()
            buf = bufs[c % NBUF]

            @plsc.parallel_loop(0, CHUNK // L, 1, unroll=UNROLL)
            def _(i):
                r = lax.shift_right_logical(i, 5)
                col = lax.shift_left(jnp.bitwise_and(i, 31), 4)
                v = jnp.bitwise_and(buf[r, pl.ds(col, L)], 255)
                buf[r, pl.ds(col, L)] = plsc.load_gather(lut, [v])

            outs[c] = out_dma(ch, c)
        for c in sorted(outs):
            outs.pop(c).wait()
        return 0

    lax.fori_loop(0, CPW, channel_body, 0)


def kernel(img):
    B, C, H, W = img.shape
    mesh = plsc.VectorSubcoreMesh(
        core_axis_name="c", subcore_axis_name="s", num_cores=2, num_subcores=16
    )
    out = pl.kernel(
        _body,
        out_type=jax.ShapeDtypeStruct((B, C, H, W), jnp.int32),
        mesh=mesh,
        scratch_types=[
            pltpu.VMEM((ROWS, 512), jnp.int32),
            pltpu.VMEM((ROWS, 512), jnp.int32),
            pltpu.VMEM((ROWS, 512), jnp.int32),
            pltpu.VMEM((ROWS, 512), jnp.int32),
            pltpu.VMEM((NBIN,), jnp.int32),
            pltpu.VMEM((NBIN + L,), jnp.int32),
            pltpu.SemaphoreType.DMA,
            pltpu.SemaphoreType.DMA,
            pltpu.SemaphoreType.DMA,
            pltpu.SemaphoreType.DMA,
            pltpu.SemaphoreType.DMA,
            pltpu.SemaphoreType.DMA,
            pltpu.SemaphoreType.DMA,
            pltpu.SemaphoreType.DMA,
        ],
        compiler_params=pltpu.CompilerParams(needs_layout_passes=False),
    )(img)
    return out
